# baseline (speedup 1.0000x reference)
import jax
import jax.numpy as jnp
from jax import lax
from jax.experimental import pallas as pl
from jax.experimental.pallas import tpu as pltpu
from jax.experimental.pallas import tpu_sc as plsc

_BATCH = 16384
_FEAT = 128
_LANES = 16
_NW = 32
_BPW = _BATCH // _NW
_CHUNKS = (64, 128, 128, 128, 64)
_OFFS = tuple(sum(_CHUNKS[:i]) for i in range(len(_CHUNKS)))
_MAXC = max(_CHUNKS)
_NACC = _FEAT // _LANES
_NBUF = 3


def _body(feat_hbm, lab_hbm, cent_hbm, out_hbm, idx_v, rows_v, feat_v,
          acc_v, gsem, fsem):
    wid = lax.axis_index("s") * 2 + lax.axis_index("c")
    base = wid * _BPW

    def issue_gather(c):
        sz = _CHUNKS[c]
        return pltpu.async_copy(
            cent_hbm.at[idx_v.at[pl.ds(_OFFS[c], sz)]],
            rows_v.at[c % _NBUF, pl.ds(0, sz)], gsem)

    nahead = _NBUF - 1
    fcopy = pltpu.async_copy(feat_hbm.at[pl.ds(base, _BPW)], feat_v, fsem)
    pltpu.sync_copy(lab_hbm.at[pl.ds(base, _BPW)], idx_v)
    gpend = [issue_gather(c) for c in range(nahead)]

    accs = (jnp.zeros((_LANES,), jnp.float32),) * _NACC
    for c in range(len(_CHUNKS)):
        g = gpend.pop(0)
        if c + nahead < len(_CHUNKS):
            gpend.append(issue_gather(c + nahead))
        g.wait()
        if c == 0:
            fcopy.wait()
        b = c % _NBUF
        rows_b = rows_v.at[b]
        feat_b = feat_v.at[pl.ds(_OFFS[c], _CHUNKS[c])]

        def row_body(r, a, rows_b=rows_b, feat_b=feat_b):
            out = list(a)
            for d in range(_NACC):
                fv = feat_b[r, pl.ds(d * _LANES, _LANES)]
                gv = rows_b[r, pl.ds(d * _LANES, _LANES)]
                df = fv - gv
                out[d] = out[d] + df * df
            return tuple(out)

        accs = plsc.parallel_loop(0, _CHUNKS[c], 1, unroll=8,
                                  carry=accs)(row_body)

    total = accs[0]
    for d in range(1, _NACC):
        total = total + accs[d]
    acc_v[...] = total
    pltpu.sync_copy(acc_v, out_hbm.at[wid])


@jax.jit
def kernel(features, labels, centers):
    mesh = plsc.VectorSubcoreMesh(core_axis_name="c", subcore_axis_name="s")
    partials = pl.kernel(
        _body,
        out_type=jax.ShapeDtypeStruct((_NW, _LANES), jnp.float32),
        mesh=mesh,
        scratch_types=[
            pltpu.VMEM((_BPW,), jnp.int32),
            pltpu.VMEM((_NBUF, _MAXC, _FEAT), jnp.float32),
            pltpu.VMEM((_BPW, _FEAT), jnp.float32),
            pltpu.VMEM((_LANES,), jnp.float32),
            pltpu.SemaphoreType.DMA,
            pltpu.SemaphoreType.DMA,
        ],
    )(features, labels.astype(jnp.int32), centers)
    return jnp.sum(partials) / (2.0 * features.shape[0])


# speedup vs baseline: 1.0394x; 1.0394x over previous
import jax
import jax.numpy as jnp
from jax import lax
from jax.experimental import pallas as pl
from jax.experimental.pallas import tpu as pltpu
from jax.experimental.pallas import tpu_sc as plsc

_BATCH = 16384
_FEAT = 128
_LANES = 16
_NW = 32
_BPW = _BATCH // _NW
_CHUNKS = (64, 128, 128, 128, 64)
_OFFS = tuple(sum(_CHUNKS[:i]) for i in range(len(_CHUNKS)))
_MAXC = max(_CHUNKS)
_NACC = _FEAT // _LANES
_NBUF = 3


def _body(feat_hbm, lab_hbm, cent_hbm, out_hbm, idx_v, rows_v, feat_v,
          acc_v, gsem, fsem):
    wid = lax.axis_index("s") * 2 + lax.axis_index("c")
    base = wid * _BPW

    def issue_gather(c):
        sz = _CHUNKS[c]
        return pltpu.async_copy(
            cent_hbm.at[idx_v.at[pl.ds(_OFFS[c], sz)]],
            rows_v.at[c % _NBUF, pl.ds(0, sz)], gsem)

    def issue_feat(c):
        sz = _CHUNKS[c]
        return pltpu.async_copy(
            feat_hbm.at[pl.ds(base + _OFFS[c], sz)],
            feat_v.at[pl.ds(_OFFS[c], sz)], fsem)

    nahead = _NBUF - 1
    fpend = [issue_feat(c) for c in range(2)]
    pltpu.sync_copy(lab_hbm.at[pl.ds(base, _BPW)], idx_v)
    gpend = [issue_gather(c) for c in range(nahead)]
    fpend += [issue_feat(c) for c in range(2, len(_CHUNKS))]

    accs = (jnp.zeros((_LANES,), jnp.float32),) * _NACC
    for c in range(len(_CHUNKS)):
        g = gpend.pop(0)
        f = fpend.pop(0)
        if c + nahead < len(_CHUNKS):
            gpend.append(issue_gather(c + nahead))
        g.wait()
        f.wait()
        b = c % _NBUF
        rows_b = rows_v.at[b]
        feat_b = feat_v.at[pl.ds(_OFFS[c], _CHUNKS[c])]

        def row_body(r, a, rows_b=rows_b, feat_b=feat_b):
            out = list(a)
            for d in range(_NACC):
                fv = feat_b[r, pl.ds(d * _LANES, _LANES)]
                gv = rows_b[r, pl.ds(d * _LANES, _LANES)]
                df = fv - gv
                out[d] = out[d] + df * df
            return tuple(out)

        accs = plsc.parallel_loop(0, _CHUNKS[c], 1, unroll=8,
                                  carry=accs)(row_body)

    total = accs[0]
    for d in range(1, _NACC):
        total = total + accs[d]
    acc_v[...] = total
    pltpu.sync_copy(acc_v, out_hbm.at[wid])


@jax.jit
def kernel(features, labels, centers):
    mesh = plsc.VectorSubcoreMesh(core_axis_name="c", subcore_axis_name="s")
    partials = pl.kernel(
        _body,
        out_type=jax.ShapeDtypeStruct((_NW, _LANES), jnp.float32),
        mesh=mesh,
        scratch_types=[
            pltpu.VMEM((_BPW,), jnp.int32),
            pltpu.VMEM((_NBUF, _MAXC, _FEAT), jnp.float32),
            pltpu.VMEM((_BPW, _FEAT), jnp.float32),
            pltpu.VMEM((_LANES,), jnp.float32),
            pltpu.SemaphoreType.DMA,
            pltpu.SemaphoreType.DMA,
        ],
    )(features, labels.astype(jnp.int32), centers)
    return jnp.sum(partials) / (2.0 * features.shape[0])


# speedup vs baseline: 1.0675x; 1.0270x over previous
import jax
import jax.numpy as jnp
from jax import lax
from jax.experimental import pallas as pl
from jax.experimental.pallas import tpu as pltpu
from jax.experimental.pallas import tpu_sc as plsc

_BATCH = 16384
_FEAT = 128
_LANES = 16
_NW = 32
_BPW = _BATCH // _NW
_CHUNKS = (32, 96, 128, 128, 96, 32)
_OFFS = tuple(sum(_CHUNKS[:i]) for i in range(len(_CHUNKS)))
_MAXC = max(_CHUNKS)
_NACC = _FEAT // _LANES
_NBUF = 3


def _body(feat_hbm, lab_hbm, cent_hbm, out_hbm, idx_v, rows_v, feat_v,
          acc_v, gsem, fsem):
    wid = lax.axis_index("s") * 2 + lax.axis_index("c")
    base = wid * _BPW

    def issue_gather(c):
        sz = _CHUNKS[c]
        return pltpu.async_copy(
            cent_hbm.at[idx_v.at[pl.ds(_OFFS[c], sz)]],
            rows_v.at[c % _NBUF, pl.ds(0, sz)], gsem)

    def issue_feat(c):
        sz = _CHUNKS[c]
        return pltpu.async_copy(
            feat_hbm.at[pl.ds(base + _OFFS[c], sz)],
            feat_v.at[c % _NBUF, pl.ds(0, sz)], fsem)

    nahead = _NBUF - 1
    fpend = [issue_feat(c) for c in range(nahead)]
    pltpu.sync_copy(lab_hbm.at[pl.ds(base, _BPW)], idx_v)
    gpend = [issue_gather(c) for c in range(nahead)]

    accs = (jnp.zeros((_LANES,), jnp.float32),) * _NACC
    for c in range(len(_CHUNKS)):
        g = gpend.pop(0)
        f = fpend.pop(0)
        if c + nahead < len(_CHUNKS):
            fpend.append(issue_feat(c + nahead))
            gpend.append(issue_gather(c + nahead))
        g.wait()
        f.wait()
        b = c % _NBUF
        rows_b = rows_v.at[b]
        feat_b = feat_v.at[b]

        def row_body(r, a, rows_b=rows_b, feat_b=feat_b):
            out = list(a)
            for d in range(_NACC):
                fv = feat_b[r, pl.ds(d * _LANES, _LANES)]
                gv = rows_b[r, pl.ds(d * _LANES, _LANES)]
                df = fv - gv
                out[d] = out[d] + df * df
            return tuple(out)

        accs = plsc.parallel_loop(0, _CHUNKS[c], 1, unroll=8,
                                  carry=accs)(row_body)

    total = accs[0]
    for d in range(1, _NACC):
        total = total + accs[d]
    acc_v[...] = total
    pltpu.sync_copy(acc_v, out_hbm.at[wid])


@jax.jit
def kernel(features, labels, centers):
    mesh = plsc.VectorSubcoreMesh(core_axis_name="c", subcore_axis_name="s")
    partials = pl.kernel(
        _body,
        out_type=jax.ShapeDtypeStruct((_NW, _LANES), jnp.float32),
        mesh=mesh,
        scratch_types=[
            pltpu.VMEM((_BPW,), jnp.int32),
            pltpu.VMEM((_NBUF, _MAXC, _FEAT), jnp.float32),
            pltpu.VMEM((_NBUF, _MAXC, _FEAT), jnp.float32),
            pltpu.VMEM((_LANES,), jnp.float32),
            pltpu.SemaphoreType.DMA,
            pltpu.SemaphoreType.DMA,
        ],
    )(features, labels.astype(jnp.int32), centers)
    return jnp.sum(partials) / (2.0 * features.shape[0])
